# baseline (speedup 1.0000x reference)
import functools

import jax
import jax.numpy as jnp
from jax import lax
from jax.experimental import pallas as pl
from jax.experimental.pallas import tpu as pltpu
from jax.experimental.pallas import tpu_sc as plsc

N_ITEMS = 100001
D = 128
DW = D // 2
B = 4096
H = 200
TOTAL = B * H
SCALE = 50.0 ** 0.5

NC = 2
NS = 16
NW = NC * NS
PER_W = TOTAL // NW
CHUNK = 128
NCHUNK = PER_W // CHUNK
NRING = 4
LANES = 16

PACK_BR = 512
NPB = (N_ITEMS + PACK_BR - 1) // PACK_BR


def _pack_body(t_ref, o_ref):
    x = t_ref[...]
    lo = lax.bitcast_convert_type(
        x[:, :DW].astype(jnp.bfloat16), jnp.uint16).astype(jnp.uint32)
    hi = lax.bitcast_convert_type(
        x[:, DW:].astype(jnp.bfloat16), jnp.uint16).astype(jnp.uint32)
    o_ref[...] = lo | (hi << 16)


def _emb_body(x_hbm, packed_hbm, out_hbm, idx_all, grows, frows,
              sem_g, sem_s):
    wid = lax.axis_index("s") * NC + lax.axis_index("c")
    base = wid * PER_W

    pltpu.sync_copy(x_hbm.at[wid], idx_all)

    def gather(j, slot):
        return pltpu.make_async_copy(
            packed_hbm.at[idx_all.at[j]],
            grows.at[pl.ds(slot * CHUNK, CHUNK)], sem_g)

    def scatter(j, slot):
        return pltpu.make_async_copy(
            frows.at[pl.ds(slot * CHUNK, CHUNK)],
            out_hbm.at[pl.ds(base + j * CHUNK, CHUNK)], sem_s)

    gather(0, 0).start()
    gather(1, 1).start()

    mask_hi = jnp.uint32(0xFFFF0000)

    def step(i, carry):
        h = lax.rem(i, NRING)
        gather(i, h).wait()

        @pl.when(i >= 2)
        def _():
            scatter(i - 2, lax.rem(i - 2, NRING)).wait()

        @pl.when(i + 2 < NCHUNK)
        def _():
            gather(i + 2, lax.rem(i + 2, NRING)).start()

        gbase = h * CHUNK
        fbase = h * CHUNK

        def unpack_row(r, c):
            for j in range(DW // LANES):
                w = grows[gbase + r, pl.ds(j * LANES, LANES)]
                lo = lax.bitcast_convert_type(w << 16, jnp.float32)
                hi = lax.bitcast_convert_type(w & mask_hi, jnp.float32)
                frows[fbase + r, pl.ds(j * LANES, LANES)] = lo * SCALE
                frows[fbase + r, pl.ds(DW + j * LANES, LANES)] = hi * SCALE
            return c

        lax.fori_loop(0, CHUNK, unpack_row, 0, unroll=2)
        scatter(i, h).start()
        return carry

    lax.fori_loop(0, NCHUNK, step, 0)

    scatter(NCHUNK - 2, lax.rem(NCHUNK - 2, NRING)).wait()
    scatter(NCHUNK - 1, lax.rem(NCHUNK - 1, NRING)).wait()


@jax.jit
def _emb(x_resh, table):
    packed = pl.pallas_call(
        _pack_body,
        grid=(NPB,),
        in_specs=[pl.BlockSpec((PACK_BR, D), lambda i: (i, 0))],
        out_specs=pl.BlockSpec((PACK_BR, DW), lambda i: (i, 0)),
        out_shape=jax.ShapeDtypeStruct((N_ITEMS, DW), jnp.uint32),
    )(table)

    mesh = plsc.VectorSubcoreMesh(core_axis_name="c", subcore_axis_name="s")
    run = functools.partial(
        pl.kernel,
        mesh=mesh,
        compiler_params=pltpu.CompilerParams(use_tc_tiling_on_sc=False),
        out_type=jax.ShapeDtypeStruct((TOTAL, D), jnp.float32),
        scratch_types=[
            pltpu.VMEM((NCHUNK, CHUNK), jnp.int32),
            pltpu.VMEM((NRING * CHUNK, DW), jnp.uint32),
            pltpu.VMEM((NRING * CHUNK, D), jnp.float32),
            pltpu.SemaphoreType.DMA,
            pltpu.SemaphoreType.DMA,
        ],
    )(_emb_body)
    return run(x_resh, packed)


def kernel(X, table):
    out = _emb(X.reshape(NW, NCHUNK, CHUNK), table)
    return out.reshape(B, H, D)


# speedup vs baseline: 2.4319x; 2.4319x over previous
import functools

import jax
import jax.numpy as jnp
from jax import lax
from jax.experimental import pallas as pl
from jax.experimental.pallas import tpu as pltpu
from jax.experimental.pallas import tpu_sc as plsc

N_ITEMS = 100001
D = 128
B = 4096
H = 200
TOTAL = B * H
SCALE = 50.0 ** 0.5

NC = 2
NS = 16
NW = NC * NS
PER_W = TOTAL // NW
CHUNK = 128
NCHUNK = PER_W // CHUNK
NRING = 6
LANES = 16


def _emb_body(x_hbm, table_hbm, out_hbm, idx_all, rows_v, sem_g, sem_s):
    wid = lax.axis_index("s") * NC + lax.axis_index("c")
    base = wid * PER_W

    pltpu.sync_copy(x_hbm.at[wid], idx_all)

    def gather(j, slot):
        return pltpu.make_async_copy(
            table_hbm.at[idx_all.at[j]],
            rows_v.at[pl.ds(slot * CHUNK, CHUNK)], sem_g)

    def scatter(j, slot):
        return pltpu.make_async_copy(
            rows_v.at[pl.ds(slot * CHUNK, CHUNK)],
            out_hbm.at[pl.ds(base + j * CHUNK, CHUNK)], sem_s)

    gather(0, 0).start()
    gather(1, 1).start()
    gather(2, 2).start()

    def step(i, carry):
        h = lax.rem(i, NRING)
        gather(i, h).wait()

        @pl.when(i >= 3)
        def _():
            scatter(i - 3, lax.rem(i - 3, NRING)).wait()

        @pl.when(i + 3 < NCHUNK)
        def _():
            gather(i + 3, lax.rem(i + 3, NRING)).start()

        rbase = h * CHUNK

        def scale_row(r, c):
            for j in range(D // LANES):
                sl = pl.ds(j * LANES, LANES)
                rows_v[rbase + r, sl] = rows_v[rbase + r, sl] * SCALE
            return c

        lax.fori_loop(0, CHUNK, scale_row, 0, unroll=4)
        scatter(i, h).start()
        return carry

    lax.fori_loop(0, NCHUNK, step, 0)

    scatter(NCHUNK - 3, lax.rem(NCHUNK - 3, NRING)).wait()
    scatter(NCHUNK - 2, lax.rem(NCHUNK - 2, NRING)).wait()
    scatter(NCHUNK - 1, lax.rem(NCHUNK - 1, NRING)).wait()


@jax.jit
def _emb(x_resh, table):
    mesh = plsc.VectorSubcoreMesh(core_axis_name="c", subcore_axis_name="s")
    run = functools.partial(
        pl.kernel,
        mesh=mesh,
        out_type=jax.ShapeDtypeStruct((TOTAL, D), jnp.float32),
        scratch_types=[
            pltpu.VMEM((NCHUNK, CHUNK), jnp.int32),
            pltpu.VMEM((NRING * CHUNK, D), jnp.float32),
            pltpu.SemaphoreType.DMA,
            pltpu.SemaphoreType.DMA,
        ],
    )(_emb_body)
    return run(x_resh, table)


def kernel(X, table):
    out = _emb(X.reshape(NW, NCHUNK, CHUNK), table)
    return out.reshape(B, H, D)
